# baseline (speedup 1.0000x reference)
import functools

import jax
import jax.numpy as jnp
from jax import lax
from jax.experimental import pallas as pl
from jax.experimental.pallas import tpu as pltpu
from jax.experimental.pallas import tpu_sc as plsc

N_ROWS = 16384
N_CODES = 1024
DIM = 256
N_CHUNKS = 2
CH_ROWS = N_ROWS // N_CHUNKS
BR = 4096
COMMIT = 0.25


def _tc_body(z_ref, w_ref, idx_ref, sum_ref, w2_ref, w2x_ref, acc_ref):
    step = pl.program_id(0)
    z = z_ref[...]

    @pl.when(step == 0)
    def _init():
        w = w_ref[...]
        acc_ref[0] = 0.0
        w2_ref[...] = jnp.sum(w * w, axis=1)[None, :]
        w2x_ref[...] = w + w

    z2 = jnp.sum(z * z, axis=1, keepdims=True)
    zw2 = lax.dot_general(z, w2x_ref[...], (((1,), (1,)), ((), ())),
                          preferred_element_type=jnp.float32)
    dist = (z2 + w2_ref[...]) - zw2
    m = jnp.min(dist, axis=1, keepdims=True)
    iota = lax.broadcasted_iota(jnp.int32, (1, N_CODES), 1).astype(jnp.float32)
    idx_f = jnp.min(jnp.where(dist == m, iota, float(N_CODES)), axis=1)
    idx_ref[...] = idx_f.astype(jnp.int32)

    acc_ref[0] += jnp.sum(m)

    @pl.when(step == pl.num_programs(0) - 1)
    def _fin():
        sum_ref[0, 0] = acc_ref[0]


@functools.cache
def _tc_argmin(chunk):
    grid = CH_ROWS // BR
    off = chunk * grid
    return pl.pallas_call(
        _tc_body,
        grid=(grid,),
        in_specs=[
            pl.BlockSpec((BR, DIM), lambda i: (i + off, 0)),
            pl.BlockSpec((N_CODES, DIM), lambda i: (0, 0)),
        ],
        out_specs=[
            pl.BlockSpec((BR,), lambda i: (i,)),
            pl.BlockSpec(memory_space=pltpu.SMEM),
        ],
        out_shape=[
            jax.ShapeDtypeStruct((CH_ROWS,), jnp.int32),
            jax.ShapeDtypeStruct((1, 1), jnp.float32),
        ],
        scratch_shapes=[pltpu.VMEM((1, N_CODES), jnp.float32),
                        pltpu.VMEM((N_CODES, DIM), jnp.float32),
                        pltpu.SMEM((1,), jnp.float32)],
    )


_SC_CORES = 2
_SC_SUBCORES = 16
_NW = _SC_CORES * _SC_SUBCORES
_B_PER_W = CH_ROWS // _NW
_CHUNK = 128
_NCH = _B_PER_W // _CHUNK


def _sc_body(w_hbm, idx_hbm, zq_hbm, idx_v, buf0, buf1, sem0, sem1):
    wid = lax.axis_index("s") * _SC_CORES + lax.axis_index("c")
    base = wid * _B_PER_W
    pltpu.sync_copy(idx_hbm.at[pl.ds(base, _B_PER_W)], idx_v)
    bufs = (buf0, buf1)
    sems = (sem0, sem1)
    copies = [None] * _NCH
    for c in range(min(2, _NCH)):
        copies[c] = pltpu.async_copy(
            w_hbm.at[idx_v.at[pl.ds(c * _CHUNK, _CHUNK)]], bufs[c % 2],
            sems[c % 2])
    for c in range(_NCH):
        copies[c].wait()
        pltpu.sync_copy(bufs[c % 2],
                        zq_hbm.at[pl.ds(base + c * _CHUNK, _CHUNK)])
        nxt = c + 2
        if nxt < _NCH:
            copies[nxt] = pltpu.async_copy(
                w_hbm.at[idx_v.at[pl.ds(nxt * _CHUNK, _CHUNK)]],
                bufs[nxt % 2], sems[nxt % 2])


@functools.cache
def _sc_gather():
    return pl.kernel(
        _sc_body,
        out_type=jax.ShapeDtypeStruct((CH_ROWS, DIM), jnp.float32),
        mesh=plsc.VectorSubcoreMesh(core_axis_name="c", subcore_axis_name="s"),
        scratch_types=[
            pltpu.VMEM((_B_PER_W,), jnp.int32),
            pltpu.VMEM((_CHUNK, DIM), jnp.float32),
            pltpu.VMEM((_CHUNK, DIM), jnp.float32),
            pltpu.SemaphoreType.DMA,
            pltpu.SemaphoreType.DMA,
        ],
    )


def kernel(z_e, W):
    zq_parts = []
    idx_parts = []
    sums = []
    for c in range(N_CHUNKS):
        idx_c, s = _tc_argmin(c)(z_e, W)
        zq_c = _sc_gather()(W, idx_c)
        zq_parts.append(zq_c)
        idx_parts.append(idx_c)
        sums.append(s[0, 0])
    z_q_st = jnp.concatenate(zq_parts, axis=0)
    indices = jnp.concatenate(idx_parts, axis=0)
    mean1 = sum(sums) / jnp.float32(N_ROWS * DIM)
    vq_loss = mean1 + jnp.float32(COMMIT) * mean1
    return (z_q_st, indices, vq_loss)


# speedup vs baseline: 1.1851x; 1.1851x over previous
import functools

import jax
import jax.numpy as jnp
from jax import lax
from jax.experimental import pallas as pl
from jax.experimental.pallas import tpu as pltpu
from jax.experimental.pallas import tpu_sc as plsc

N_ROWS = 16384
N_CODES = 1024
DIM = 256
N_CHUNKS = 2
CH_ROWS = N_ROWS // N_CHUNKS
BR = 4096
COMMIT = 0.25


def _tc_body(z_ref, w_ref, idx_ref, sum_ref, w2_ref, w2x_ref, acc_ref):
    step = pl.program_id(0)
    z = z_ref[...]

    @pl.when(step == 0)
    def _init():
        w = w_ref[...]
        acc_ref[0] = 0.0
        w2_ref[...] = jnp.sum(w * w, axis=1)[None, :]
        w2x_ref[...] = w + w

    z2 = jnp.sum(z * z, axis=1, keepdims=True)
    zw2 = lax.dot_general(z, w2x_ref[...], (((1,), (1,)), ((), ())),
                          preferred_element_type=jnp.float32)
    dist = (z2 + w2_ref[...]) - zw2
    m = jnp.min(dist, axis=1, keepdims=True)
    iota = lax.broadcasted_iota(jnp.int32, (1, N_CODES), 1).astype(jnp.float32)
    idx_f = jnp.min(jnp.where(dist == m, iota, float(N_CODES)),
                    axis=1, keepdims=True)
    idx_ref[...] = jnp.transpose(idx_f.astype(jnp.int32), (1, 0))

    acc_ref[0] += jnp.sum(m)

    @pl.when(step == pl.num_programs(0) - 1)
    def _fin():
        sum_ref[0, 0] = acc_ref[0]


@functools.cache
def _tc_argmin(chunk):
    grid = CH_ROWS // BR
    off = chunk * grid
    return pl.pallas_call(
        _tc_body,
        grid=(grid,),
        in_specs=[
            pl.BlockSpec((BR, DIM), lambda i: (i + off, 0)),
            pl.BlockSpec((N_CODES, DIM), lambda i: (0, 0)),
        ],
        out_specs=[
            pl.BlockSpec((1, BR), lambda i: (0, i)),
            pl.BlockSpec(memory_space=pltpu.SMEM),
        ],
        out_shape=[
            jax.ShapeDtypeStruct((1, CH_ROWS), jnp.int32),
            jax.ShapeDtypeStruct((1, 1), jnp.float32),
        ],
        scratch_shapes=[pltpu.VMEM((1, N_CODES), jnp.float32),
                        pltpu.VMEM((N_CODES, DIM), jnp.float32),
                        pltpu.SMEM((1,), jnp.float32)],
    )


_SC_CORES = 2
_SC_SUBCORES = 16
_NW = _SC_CORES * _SC_SUBCORES
_B_PER_W = CH_ROWS // _NW
_CHUNK = 128
_NCH = _B_PER_W // _CHUNK


def _sc_body(w_hbm, idx_hbm, zq_hbm, idx_v, buf0, buf1, sem0, sem1):
    wid = lax.axis_index("s") * _SC_CORES + lax.axis_index("c")
    base = wid * _B_PER_W
    pltpu.sync_copy(idx_hbm.at[pl.ds(base, _B_PER_W)], idx_v)
    bufs = (buf0, buf1)
    sems = (sem0, sem1)
    copies = [None] * _NCH
    for c in range(min(2, _NCH)):
        copies[c] = pltpu.async_copy(
            w_hbm.at[idx_v.at[pl.ds(c * _CHUNK, _CHUNK)]],
            bufs[c % 2], sems[c % 2])
    for c in range(_NCH):
        copies[c].wait()
        pltpu.sync_copy(bufs[c % 2],
                        zq_hbm.at[pl.ds(base + c * _CHUNK, _CHUNK)])
        nxt = c + 2
        if nxt < _NCH:
            copies[nxt] = pltpu.async_copy(
                w_hbm.at[idx_v.at[pl.ds(nxt * _CHUNK, _CHUNK)]],
                bufs[nxt % 2], sems[nxt % 2])


@functools.cache
def _sc_gather():
    return pl.kernel(
        _sc_body,
        out_type=jax.ShapeDtypeStruct((CH_ROWS, DIM), jnp.float32),
        mesh=plsc.VectorSubcoreMesh(core_axis_name="c", subcore_axis_name="s"),
        scratch_types=[
            pltpu.VMEM((_B_PER_W,), jnp.int32),
            pltpu.VMEM((_CHUNK, DIM), jnp.float32),
            pltpu.VMEM((_CHUNK, DIM), jnp.float32),
            pltpu.SemaphoreType.DMA,
            pltpu.SemaphoreType.DMA,
        ],
    )


def kernel(z_e, W):
    zq_parts = []
    idx_parts = []
    sums = []
    for c in range(N_CHUNKS):
        idxrow, s = _tc_argmin(c)(z_e, W)
        idx_c = idxrow.reshape(CH_ROWS)
        zq_c = _sc_gather()(W, idx_c)
        zq_parts.append(zq_c)
        idx_parts.append(idx_c)
        sums.append(s[0, 0])
    z_q_st = jnp.concatenate(zq_parts, axis=0)
    indices = jnp.concatenate(idx_parts, axis=0)
    mean1 = sum(sums) / jnp.float32(N_ROWS * DIM)
    vq_loss = mean1 + jnp.float32(COMMIT) * mean1
    return (z_q_st, indices, vq_loss)


# speedup vs baseline: 1.3628x; 1.1500x over previous
import functools

import jax
import jax.numpy as jnp
from jax import lax
from jax.experimental import pallas as pl
from jax.experimental.pallas import tpu as pltpu
from jax.experimental.pallas import tpu_sc as plsc

N_ROWS = 16384
N_CODES = 1024
DIM = 256
N_CHUNKS = 1
CH_ROWS = N_ROWS // N_CHUNKS
BR = 4096
COMMIT = 0.25


def _tc_body(z_ref, w_ref, idx_ref, sum_ref, w2_ref, w2x_ref, acc_ref):
    step = pl.program_id(0)
    z = z_ref[...]

    @pl.when(step == 0)
    def _init():
        w = w_ref[...]
        acc_ref[0] = 0.0
        w2_ref[...] = jnp.sum(w * w, axis=1)[None, :]
        w2x_ref[...] = w + w

    z2 = jnp.sum(z * z, axis=1, keepdims=True)
    zw2 = lax.dot_general(z, w2x_ref[...], (((1,), (1,)), ((), ())),
                          preferred_element_type=jnp.float32)
    dist = (z2 + w2_ref[...]) - zw2
    m = jnp.min(dist, axis=1, keepdims=True)
    iota = lax.broadcasted_iota(jnp.int32, (1, N_CODES), 1).astype(jnp.float32)
    idx_f = jnp.min(jnp.where(dist == m, iota, float(N_CODES)),
                    axis=1, keepdims=True)
    idx_ref[...] = jnp.transpose(idx_f.astype(jnp.int32), (1, 0))

    acc_ref[0] += jnp.sum(m)

    @pl.when(step == pl.num_programs(0) - 1)
    def _fin():
        sum_ref[0, 0] = acc_ref[0]


@functools.cache
def _tc_argmin(chunk):
    grid = CH_ROWS // BR
    off = chunk * grid
    return pl.pallas_call(
        _tc_body,
        grid=(grid,),
        in_specs=[
            pl.BlockSpec((BR, DIM), lambda i: (i + off, 0)),
            pl.BlockSpec((N_CODES, DIM), lambda i: (0, 0)),
        ],
        out_specs=[
            pl.BlockSpec((1, BR), lambda i: (0, i)),
            pl.BlockSpec(memory_space=pltpu.SMEM),
        ],
        out_shape=[
            jax.ShapeDtypeStruct((1, CH_ROWS), jnp.int32),
            jax.ShapeDtypeStruct((1, 1), jnp.float32),
        ],
        scratch_shapes=[pltpu.VMEM((1, N_CODES), jnp.float32),
                        pltpu.VMEM((N_CODES, DIM), jnp.float32),
                        pltpu.SMEM((1,), jnp.float32)],
    )


_SC_CORES = 2
_SC_SUBCORES = 16
_NW = _SC_CORES * _SC_SUBCORES
_B_PER_W = CH_ROWS // _NW
_CHUNK = 128
_NCH = _B_PER_W // _CHUNK


def _sc_body(w_hbm, idx_hbm, zq_hbm, idx_v, buf0, buf1, sem0, sem1):
    wid = lax.axis_index("s") * _SC_CORES + lax.axis_index("c")
    base = wid * _B_PER_W
    pltpu.sync_copy(idx_hbm.at[pl.ds(base, _B_PER_W)], idx_v)
    bufs = (buf0, buf1)
    sems = (sem0, sem1)
    copies = [None] * _NCH
    for c in range(min(2, _NCH)):
        copies[c] = pltpu.async_copy(
            w_hbm.at[idx_v.at[pl.ds(c * _CHUNK, _CHUNK)]],
            bufs[c % 2], sems[c % 2])
    for c in range(_NCH):
        copies[c].wait()
        pltpu.sync_copy(bufs[c % 2],
                        zq_hbm.at[pl.ds(base + c * _CHUNK, _CHUNK)])
        nxt = c + 2
        if nxt < _NCH:
            copies[nxt] = pltpu.async_copy(
                w_hbm.at[idx_v.at[pl.ds(nxt * _CHUNK, _CHUNK)]],
                bufs[nxt % 2], sems[nxt % 2])


@functools.cache
def _sc_gather():
    return pl.kernel(
        _sc_body,
        out_type=jax.ShapeDtypeStruct((CH_ROWS, DIM), jnp.float32),
        mesh=plsc.VectorSubcoreMesh(core_axis_name="c", subcore_axis_name="s"),
        scratch_types=[
            pltpu.VMEM((_B_PER_W,), jnp.int32),
            pltpu.VMEM((_CHUNK, DIM), jnp.float32),
            pltpu.VMEM((_CHUNK, DIM), jnp.float32),
            pltpu.SemaphoreType.DMA,
            pltpu.SemaphoreType.DMA,
        ],
    )


def kernel(z_e, W):
    idxrow, s = _tc_argmin(0)(z_e, W)
    indices = idxrow.reshape(N_ROWS)
    z_q_st = _sc_gather()(W, indices)
    mean1 = s[0, 0] / jnp.float32(N_ROWS * DIM)
    vq_loss = mean1 + jnp.float32(COMMIT) * mean1
    return (z_q_st, indices, vq_loss)


# speedup vs baseline: 1.3630x; 1.0001x over previous
import functools

import jax
import jax.numpy as jnp
from jax import lax
from jax.experimental import pallas as pl
from jax.experimental.pallas import tpu as pltpu
from jax.experimental.pallas import tpu_sc as plsc

N_ROWS = 16384
N_CODES = 1024
DIM = 256
N_CHUNKS = 1
CH_ROWS = N_ROWS // N_CHUNKS
BR = 8192
COMMIT = 0.25


def _tc_body(z_ref, w_ref, idx_ref, sum_ref, w2_ref, w2x_ref, acc_ref):
    step = pl.program_id(0)
    z = z_ref[...]

    @pl.when(step == 0)
    def _init():
        w = w_ref[...]
        acc_ref[0] = 0.0
        w2_ref[...] = jnp.sum(w * w, axis=1)[None, :]
        w2x_ref[...] = w + w

    z2 = jnp.sum(z * z, axis=1, keepdims=True)
    zw2 = lax.dot_general(z, w2x_ref[...], (((1,), (1,)), ((), ())),
                          preferred_element_type=jnp.float32)
    dist = (z2 + w2_ref[...]) - zw2
    m = jnp.min(dist, axis=1, keepdims=True)
    iota = lax.broadcasted_iota(jnp.int32, (1, N_CODES), 1).astype(jnp.float32)
    idx_f = jnp.min(jnp.where(dist == m, iota, float(N_CODES)),
                    axis=1, keepdims=True)
    idx_ref[...] = jnp.transpose(idx_f.astype(jnp.int32), (1, 0))

    acc_ref[0] += jnp.sum(m)

    @pl.when(step == pl.num_programs(0) - 1)
    def _fin():
        sum_ref[0, 0] = acc_ref[0]


@functools.cache
def _tc_argmin(chunk):
    grid = CH_ROWS // BR
    off = chunk * grid
    return pl.pallas_call(
        _tc_body,
        grid=(grid,),
        in_specs=[
            pl.BlockSpec((BR, DIM), lambda i: (i + off, 0)),
            pl.BlockSpec((N_CODES, DIM), lambda i: (0, 0)),
        ],
        out_specs=[
            pl.BlockSpec((1, BR), lambda i: (0, i)),
            pl.BlockSpec(memory_space=pltpu.SMEM),
        ],
        out_shape=[
            jax.ShapeDtypeStruct((1, CH_ROWS), jnp.int32),
            jax.ShapeDtypeStruct((1, 1), jnp.float32),
        ],
        scratch_shapes=[pltpu.VMEM((1, N_CODES), jnp.float32),
                        pltpu.VMEM((N_CODES, DIM), jnp.float32),
                        pltpu.SMEM((1,), jnp.float32)],
    )


_SC_CORES = 2
_SC_SUBCORES = 16
_NW = _SC_CORES * _SC_SUBCORES
_B_PER_W = CH_ROWS // _NW
_CHUNK = 128
_NCH = _B_PER_W // _CHUNK


def _sc_body(w_hbm, idx_hbm, zq_hbm, idx_v, buf0, buf1, sem0, sem1):
    wid = lax.axis_index("s") * _SC_CORES + lax.axis_index("c")
    base = wid * _B_PER_W
    pltpu.sync_copy(idx_hbm.at[pl.ds(base, _B_PER_W)], idx_v)
    bufs = (buf0, buf1)
    sems = (sem0, sem1)
    copies = [None] * _NCH
    for c in range(min(2, _NCH)):
        copies[c] = pltpu.async_copy(
            w_hbm.at[idx_v.at[pl.ds(c * _CHUNK, _CHUNK)]],
            bufs[c % 2], sems[c % 2])
    for c in range(_NCH):
        copies[c].wait()
        pltpu.sync_copy(bufs[c % 2],
                        zq_hbm.at[pl.ds(base + c * _CHUNK, _CHUNK)])
        nxt = c + 2
        if nxt < _NCH:
            copies[nxt] = pltpu.async_copy(
                w_hbm.at[idx_v.at[pl.ds(nxt * _CHUNK, _CHUNK)]],
                bufs[nxt % 2], sems[nxt % 2])


@functools.cache
def _sc_gather():
    return pl.kernel(
        _sc_body,
        out_type=jax.ShapeDtypeStruct((CH_ROWS, DIM), jnp.float32),
        mesh=plsc.VectorSubcoreMesh(core_axis_name="c", subcore_axis_name="s"),
        scratch_types=[
            pltpu.VMEM((_B_PER_W,), jnp.int32),
            pltpu.VMEM((_CHUNK, DIM), jnp.float32),
            pltpu.VMEM((_CHUNK, DIM), jnp.float32),
            pltpu.SemaphoreType.DMA,
            pltpu.SemaphoreType.DMA,
        ],
    )


def kernel(z_e, W):
    idxrow, s = _tc_argmin(0)(z_e, W)
    indices = idxrow.reshape(N_ROWS)
    z_q_st = _sc_gather()(W, indices)
    mean1 = s[0, 0] / jnp.float32(N_ROWS * DIM)
    vq_loss = mean1 + jnp.float32(COMMIT) * mean1
    return (z_q_st, indices, vq_loss)


# speedup vs baseline: 1.3679x; 1.0036x over previous
import functools

import jax
import jax.numpy as jnp
from jax import lax
from jax.experimental import pallas as pl
from jax.experimental.pallas import tpu as pltpu
from jax.experimental.pallas import tpu_sc as plsc

N_ROWS = 16384
N_CODES = 1024
DIM = 256
N_CHUNKS = 1
CH_ROWS = N_ROWS // N_CHUNKS
BR = 4096
COMMIT = 0.25


def _tc_body(z_ref, w_ref, idx_ref, sum_ref, w2_ref, w2x_ref, acc_ref):
    step = pl.program_id(0)
    z = z_ref[...]

    @pl.when(step == 0)
    def _init():
        w = w_ref[...]
        acc_ref[0] = 0.0
        w2_ref[...] = jnp.sum(w * w, axis=1)[None, :]
        w2x_ref[...] = w + w

    z2 = jnp.sum(z * z, axis=1, keepdims=True)
    zw2 = lax.dot_general(z, w2x_ref[...], (((1,), (1,)), ((), ())),
                          preferred_element_type=jnp.float32)
    dist = (z2 + w2_ref[...]) - zw2
    m = jnp.min(dist, axis=1, keepdims=True)
    iota = lax.broadcasted_iota(jnp.int32, (1, N_CODES), 1).astype(jnp.float32)
    idx_f = jnp.min(jnp.where(dist == m, iota, float(N_CODES)),
                    axis=1, keepdims=True)
    idx_ref[...] = jnp.transpose(idx_f.astype(jnp.int32), (1, 0))

    acc_ref[0] += jnp.sum(m)

    @pl.when(step == pl.num_programs(0) - 1)
    def _fin():
        sum_ref[0, 0] = acc_ref[0]


@functools.cache
def _tc_argmin(chunk):
    grid = CH_ROWS // BR
    off = chunk * grid
    return pl.pallas_call(
        _tc_body,
        grid=(grid,),
        in_specs=[
            pl.BlockSpec((BR, DIM), lambda i: (i + off, 0)),
            pl.BlockSpec((N_CODES, DIM), lambda i: (0, 0)),
        ],
        out_specs=[
            pl.BlockSpec((1, BR), lambda i: (0, i)),
            pl.BlockSpec(memory_space=pltpu.SMEM),
        ],
        out_shape=[
            jax.ShapeDtypeStruct((1, CH_ROWS), jnp.int32),
            jax.ShapeDtypeStruct((1, 1), jnp.float32),
        ],
        scratch_shapes=[pltpu.VMEM((1, N_CODES), jnp.float32),
                        pltpu.VMEM((N_CODES, DIM), jnp.float32),
                        pltpu.SMEM((1,), jnp.float32)],
    )


_SC_CORES = 2
_SC_SUBCORES = 16
_NW = _SC_CORES * _SC_SUBCORES
_B_PER_W = CH_ROWS // _NW
_CHUNK = 128
_NCH = _B_PER_W // _CHUNK


_NBUF = 3


def _sc_body(w_hbm, idx_hbm, zq_hbm, idx_v, b0, b1, b2, g0, g1, g2,
             s0, s1, s2):
    wid = lax.axis_index("s") * _SC_CORES + lax.axis_index("c")
    base = wid * _B_PER_W
    pltpu.sync_copy(idx_hbm.at[pl.ds(base, _B_PER_W)], idx_v)
    bufs = (b0, b1, b2)
    gsems = (g0, g1, g2)
    wsems = (s0, s1, s2)
    gathers = [None] * _NCH
    writes = [None] * _NCH
    for c in range(min(_NBUF, _NCH)):
        gathers[c] = pltpu.async_copy(
            w_hbm.at[idx_v.at[pl.ds(c * _CHUNK, _CHUNK)]],
            bufs[c % _NBUF], gsems[c % _NBUF])
    for c in range(_NCH):
        gathers[c].wait()
        writes[c] = pltpu.async_copy(
            bufs[c % _NBUF],
            zq_hbm.at[pl.ds(base + c * _CHUNK, _CHUNK)],
            wsems[c % _NBUF])
        nxt = c + _NBUF
        if nxt < _NCH:
            writes[c].wait()
            gathers[nxt] = pltpu.async_copy(
                w_hbm.at[idx_v.at[pl.ds(nxt * _CHUNK, _CHUNK)]],
                bufs[nxt % _NBUF], gsems[nxt % _NBUF])
    for c in range(max(0, _NCH - _NBUF), _NCH):
        writes[c].wait()


@functools.cache
def _sc_gather():
    return pl.kernel(
        _sc_body,
        out_type=jax.ShapeDtypeStruct((CH_ROWS, DIM), jnp.float32),
        mesh=plsc.VectorSubcoreMesh(core_axis_name="c", subcore_axis_name="s"),
        scratch_types=[
            pltpu.VMEM((_B_PER_W,), jnp.int32),
            pltpu.VMEM((_CHUNK, DIM), jnp.float32),
            pltpu.VMEM((_CHUNK, DIM), jnp.float32),
            pltpu.VMEM((_CHUNK, DIM), jnp.float32),
            pltpu.SemaphoreType.DMA,
            pltpu.SemaphoreType.DMA,
            pltpu.SemaphoreType.DMA,
            pltpu.SemaphoreType.DMA,
            pltpu.SemaphoreType.DMA,
            pltpu.SemaphoreType.DMA,
        ],
    )


def kernel(z_e, W):
    idxrow, s = _tc_argmin(0)(z_e, W)
    indices = idxrow.reshape(N_ROWS)
    z_q_st = _sc_gather()(W, indices)
    mean1 = s[0, 0] / jnp.float32(N_ROWS * DIM)
    vq_loss = mean1 + jnp.float32(COMMIT) * mean1
    return (z_q_st, indices, vq_loss)
